# baseline (speedup 1.0000x reference)
import jax
import jax.numpy as jnp
from jax import lax
from jax.experimental import pallas as pl
from jax.experimental.pallas import tpu as pltpu
from jax.experimental.pallas import tpu_sc as plsc

_NC = 2
_NS = 16
_NW = _NC * _NS
_G = 8
_NBUF = 8

_LOG1P_C = (
    9.0837865e-08, 0.9999915, -0.49980116, 0.331334, -0.23919071,
    0.1647835, -0.09231377, 0.034418594, -0.0060748775,
)


def _log1p_poly(u):
    acc = jnp.full_like(u, _LOG1P_C[-1])
    for c in _LOG1P_C[-2::-1]:
        acc = acc * u + c
    return acc


def _sc_body(z_hbm, cond_hbm, out_hbm, buf, cond_v, in_sems, out_sems):
    N, K = z_hbm.shape
    rows_per_w = N // _NW
    nch = rows_per_w // _G
    nfull = K // 16
    tail = K - (nfull - 1) * 16 - 16
    toff = K - 16

    wid = lax.axis_index("s") * _NC + lax.axis_index("c")
    base = wid * rows_per_w

    pltpu.sync_copy(cond_hbm.at[pl.ds(base, rows_per_w)], cond_v)

    def in_copy(g, b):
        return pltpu.make_async_copy(
            z_hbm.at[pl.ds(base + g * _G, _G), :], buf.at[b], in_sems.at[b]
        )

    def out_copy(g, b):
        return pltpu.make_async_copy(
            buf.at[b], out_hbm.at[pl.ds(base + g * _G, _G), :], out_sems.at[b]
        )

    lane = lax.iota(jnp.int32, 16)
    mask0 = lane == 0
    tail_sel = lane >= (16 - tail)
    neg_inf16 = jnp.full((16,), -jnp.inf, jnp.float32)

    for g in range(_NBUF):
        in_copy(g, g).start()

    for g in range(nch):
        b = g % _NBUF
        bufb = buf.at[b]
        in_copy(g, b).wait()

        def row_body(s, carry, bufb=bufb, g=g):
            c_vec = plsc.load_gather(
                cond_v, [jnp.full((16,), g * _G + s, jnp.int32)]
            )
            c_s = jnp.max(c_vec)
            srow = jnp.full((16,), s, jnp.int32)

            @pl.when(c_s != K)
            def _():
                t_vec = plsc.load_gather(bufb, [srow, c_vec])
                plsc.store_scatter(bufb, [srow, c_vec], neg_inf16, mask=mask0)

                def mx(j, acc):
                    return jnp.maximum(acc, bufb[s, pl.ds(j * 16, 16)])

                acc = lax.fori_loop(0, nfull, mx, neg_inf16)
                acc = jnp.maximum(acc, bufb[s, pl.ds(toff, 16)])
                m2 = jnp.maximum(jnp.max(acc), jnp.float32(0.0))
                mv = jnp.full((16,), m2, jnp.float32)
                hi = jnp.maximum(t_vec, mv)
                lo = jnp.minimum(t_vec, mv)
                v = hi + _log1p_poly(jnp.exp(lo - hi))
                plsc.store_scatter(bufb, [srow, c_vec], v, mask=mask0)

            @pl.when(c_s == K)
            def _():
                def sp(j, carry2):
                    x = bufb[s, pl.ds(j * 16, 16)]
                    y = jnp.minimum(x, 0.0) - _log1p_poly(jnp.exp(-jnp.abs(x)))
                    bufb[s, pl.ds(j * 16, 16)] = y
                    return carry2

                lax.fori_loop(0, nfull, sp, 0)
                x = bufb[s, pl.ds(toff, 16)]
                y = jnp.minimum(x, 0.0) - _log1p_poly(jnp.exp(-jnp.abs(x)))
                bufb[s, pl.ds(toff, 16)] = jnp.where(tail_sel, y, x)

            return carry

        lax.fori_loop(0, _G, row_body, 0)
        out_copy(g, b).start()

        p = g + 4
        if _NBUF <= p < nch:
            pb = p % _NBUF
            out_copy(p - _NBUF, pb).wait()
            in_copy(p, pb).start()

    for g in range(nch - _NBUF, nch):
        out_copy(g, g % _NBUF).wait()


def kernel(z, cond):
    N, K = z.shape
    mesh = plsc.VectorSubcoreMesh(
        core_axis_name="c", subcore_axis_name="s", num_cores=_NC,
        num_subcores=_NS,
    )
    f = pl.kernel(
        _sc_body,
        out_type=jax.ShapeDtypeStruct((N, K), z.dtype),
        mesh=mesh,
        scratch_types=[
            pltpu.VMEM((_NBUF, _G, K), jnp.float32),
            pltpu.VMEM((N // _NW,), jnp.int32),
            pltpu.SemaphoreType.DMA((_NBUF,)),
            pltpu.SemaphoreType.DMA((_NBUF,)),
        ],
        compiler_params=pltpu.CompilerParams(needs_layout_passes=False),
    )
    return f(z, cond)
